# Initial kernel scaffold; baseline (speedup 1.0000x reference)
import jax
import jax.numpy as jnp
from jax.experimental import pallas as pl


def kernel(x, edge_index, batch, W1, b1, W2, b2):
    raise NotImplementedError("write your pallas kernel here")


# speedup vs baseline: 91.9851x; 91.9851x over previous
import functools

import jax
import jax.numpy as jnp
from jax import lax
from jax.experimental import pallas as pl
from jax.experimental.pallas import tpu as pltpu
from jax.experimental.pallas import tpu_sc as plsc

N = 50000
E = 800000
D_IN = 75
G = 1024

NC = 2
NS = 16
NW = NC * NS

N_PAD = 53248
SLICE_R = N_PAD // NS

E_PAD = 819200
EPW = E_PAD // NW
EROWS = EPW // 128
CHUNK = 3200
NCHUNK = EPW // CHUNK
ROWS = CHUNK // 128

SLICE_P = 2048
NW_P = N_PAD // SLICE_P
PROWS = SLICE_P // 128

G_PAD = 2048
GSLICE = G_PAD // NS

_mesh = plsc.VectorSubcoreMesh(core_axis_name="c", subcore_axis_name="s")
_Z16 = functools.partial(jnp.zeros, (16,), jnp.float32)


def _round_body(u_hbm, src_hbm, dst_hbm, acc_out,
                u_local, src_buf, dst_buf, val_buf, slice_buf, acc_sh):
    c = lax.axis_index("c")
    s = lax.axis_index("s")
    wid = c * NS + s

    pltpu.sync_copy(u_hbm, u_local)
    pltpu.sync_copy(src_hbm.at[pl.ds(wid * EPW, EPW)], src_buf)
    pltpu.sync_copy(dst_hbm.at[pl.ds(wid * EROWS, EROWS)], dst_buf)

    def zbody(i, carry):
        slice_buf[pl.ds(pl.multiple_of(i * 16, 16), 16)] = _Z16()
        return carry

    lax.fori_loop(0, SLICE_R // 16, zbody, 0)
    pltpu.sync_copy(slice_buf, acc_sh.at[pl.ds(s * SLICE_R, SLICE_R)])
    plsc.subcore_barrier()

    for k in range(NCHUNK):
        def gbody(j, carry):
            off = pl.multiple_of(j * 16, 16)
            idx = src_buf[pl.ds(k * CHUNK + off, 16)]
            val_buf[pl.ds(off, 16)] = plsc.load_gather(u_local, [idx])
            return carry

        lax.fori_loop(0, CHUNK // 16, gbody, 0)

        for j in range(ROWS):
            pltpu.sync_copy(val_buf.at[pl.ds(j * 128, 128)],
                            acc_sh.at[dst_buf.at[k * ROWS + j]],
                            add=True)

    plsc.subcore_barrier()
    pltpu.sync_copy(acc_sh.at[pl.ds(s * SLICE_R, SLICE_R)],
                    acc_out.at[pl.ds(c * N_PAD + s * SLICE_R, SLICE_R)])


_round = pl.kernel(
    _round_body,
    out_type=jax.ShapeDtypeStruct((NC * N_PAD,), jnp.float32),
    mesh=_mesh,
    compiler_params=pltpu.CompilerParams(needs_layout_passes=False),
    scratch_types=[
        pltpu.VMEM((N_PAD,), jnp.float32),
        pltpu.VMEM((EPW,), jnp.int32),
        pltpu.VMEM((EROWS, 128), jnp.int32),
        pltpu.VMEM((CHUNK,), jnp.float32),
        pltpu.VMEM((SLICE_R,), jnp.float32),
        pltpu.VMEM_SHARED((N_PAD,), jnp.float32),
    ],
)


def _pool_body(acc2_hbm, u1_hbm, dinv_hbm, batch_hbm, c1_hbm, pooled_out,
               a0_buf, a1_buf, u_buf, d_buf, val_buf, batch_buf, c1_buf,
               pz_buf, pool_sh):
    c = lax.axis_index("c")
    s = lax.axis_index("s")
    wid = c * NS + s

    def zbody(i, carry):
        pz_buf[pl.ds(pl.multiple_of(i * 16, 16), 16)] = _Z16()
        return carry

    lax.fori_loop(0, GSLICE // 16, zbody, 0)
    pltpu.sync_copy(pz_buf, pool_sh.at[pl.ds(s * GSLICE, GSLICE)])

    @pl.when(wid < NW_P)
    def _():
        nb = wid * SLICE_P
        pltpu.sync_copy(acc2_hbm.at[pl.ds(nb, SLICE_P)], a0_buf)
        pltpu.sync_copy(acc2_hbm.at[pl.ds(N_PAD + nb, SLICE_P)], a1_buf)
        pltpu.sync_copy(u1_hbm.at[pl.ds(nb, SLICE_P)], u_buf)
        pltpu.sync_copy(dinv_hbm.at[pl.ds(nb, SLICE_P)], d_buf)
        pltpu.sync_copy(batch_hbm.at[pl.ds(wid * PROWS, PROWS)], batch_buf)
        pltpu.sync_copy(c1_hbm.at[pl.ds(0, 16)], c1_buf)

        c1v = c1_buf[...]

        def ebody(i, carry):
            off = pl.multiple_of(i * 16, 16)
            a0 = a0_buf[pl.ds(off, 16)]
            a1 = a1_buf[pl.ds(off, 16)]
            u = u_buf[pl.ds(off, 16)]
            d = d_buf[pl.ds(off, 16)]
            val_buf[pl.ds(off, 16)] = d * (a0 + a1 + u) + c1v
            return carry

        lax.fori_loop(0, SLICE_P // 16, ebody, 0)

    plsc.subcore_barrier()

    @pl.when(wid < NW_P)
    def _():
        for j in range(PROWS):
            pltpu.sync_copy(val_buf.at[pl.ds(j * 128, 128)],
                            pool_sh.at[batch_buf.at[j]],
                            add=True)

    plsc.subcore_barrier()
    pltpu.sync_copy(pool_sh.at[pl.ds(s * GSLICE, GSLICE)],
                    pooled_out.at[pl.ds(c * G_PAD + s * GSLICE, GSLICE)])


_pool = pl.kernel(
    _pool_body,
    out_type=jax.ShapeDtypeStruct((NC * G_PAD,), jnp.float32),
    mesh=_mesh,
    compiler_params=pltpu.CompilerParams(needs_layout_passes=False),
    scratch_types=[
        pltpu.VMEM((SLICE_P,), jnp.float32),
        pltpu.VMEM((SLICE_P,), jnp.float32),
        pltpu.VMEM((SLICE_P,), jnp.float32),
        pltpu.VMEM((SLICE_P,), jnp.float32),
        pltpu.VMEM((SLICE_P,), jnp.float32),
        pltpu.VMEM((PROWS, 128), jnp.int32),
        pltpu.VMEM((16,), jnp.float32),
        pltpu.VMEM((GSLICE,), jnp.float32),
        pltpu.VMEM_SHARED((G_PAD,), jnp.float32),
    ],
)


def _matvec(x_pad, W1, W2, b1):
    blk = N_PAD // 16

    def body(x_ref, w1_ref, w2_ref, b1_ref, y_ref, c1_ref):
        w = jnp.dot(w1_ref[...], w2_ref[...],
                    preferred_element_type=jnp.float32)
        y_ref[...] = jnp.dot(x_ref[...], w,
                             preferred_element_type=jnp.float32)
        c1 = jnp.sum(b1_ref[...] * w2_ref[...][:, 0])
        c1_ref[...] = jnp.full((1, 128), c1, jnp.float32)

    return pl.pallas_call(
        body,
        grid=(16,),
        in_specs=[
            pl.BlockSpec((blk, D_IN), lambda i: (i, 0)),
            pl.BlockSpec((D_IN, 128), lambda i: (0, 0)),
            pl.BlockSpec((128, 1), lambda i: (0, 0)),
            pl.BlockSpec((1, 128), lambda i: (0, 0)),
        ],
        out_specs=[
            pl.BlockSpec((blk, 1), lambda i: (i, 0)),
            pl.BlockSpec((1, 128), lambda i: (0, 0)),
        ],
        out_shape=[
            jax.ShapeDtypeStruct((N_PAD, 1), jnp.float32),
            jax.ShapeDtypeStruct((1, 128), jnp.float32),
        ],
    )(x_pad, W1, W2, b1.reshape(1, 128))


def _e1(accD, y2):
    def body(a_ref, y_ref, dinv_ref, u0_ref):
        deg = a_ref[...][0] + a_ref[...][1] + 1.0
        dinv = lax.rsqrt(deg)
        dinv_ref[...] = dinv
        u0_ref[...] = dinv * y_ref[...]

    rows = N_PAD // 128
    blk = 32

    return pl.pallas_call(
        body,
        grid=(rows // blk,),
        in_specs=[
            pl.BlockSpec((2, blk, 128), lambda i: (0, i, 0)),
            pl.BlockSpec((blk, 128), lambda i: (i, 0)),
        ],
        out_specs=[
            pl.BlockSpec((blk, 128), lambda i: (i, 0)),
            pl.BlockSpec((blk, 128), lambda i: (i, 0)),
        ],
        out_shape=[
            jax.ShapeDtypeStruct((rows, 128), jnp.float32),
            jax.ShapeDtypeStruct((rows, 128), jnp.float32),
        ],
    )(accD.reshape(2, rows, 128), y2)


def _e2(acc1, u0_2, dinv2):
    def body(a_ref, u_ref, d_ref, u1_ref):
        d = d_ref[...]
        u1_ref[...] = d * d * (a_ref[...][0] + a_ref[...][1] + u_ref[...])

    rows = N_PAD // 128
    blk = 32

    return pl.pallas_call(
        body,
        grid=(rows // blk,),
        in_specs=[
            pl.BlockSpec((2, blk, 128), lambda i: (0, i, 0)),
            pl.BlockSpec((blk, 128), lambda i: (i, 0)),
            pl.BlockSpec((blk, 128), lambda i: (i, 0)),
        ],
        out_specs=pl.BlockSpec((blk, 128), lambda i: (i, 0)),
        out_shape=jax.ShapeDtypeStruct((rows, 128), jnp.float32),
    )(acc1.reshape(2, rows, 128), u0_2, dinv2)


def _final(pooled, b2):
    def body(p_ref, b2_ref, o_ref):
        p = p_ref[...]
        o_ref[...] = p[0, :8] + p[1, :8] + b2_ref[...]

    return pl.pallas_call(
        body,
        in_specs=[
            pl.BlockSpec((2, 16, 128), lambda: (0, 0, 0)),
            pl.BlockSpec((1, 128), lambda: (0, 0)),
        ],
        out_specs=pl.BlockSpec((8, 128), lambda: (0, 0)),
        out_shape=jax.ShapeDtypeStruct((8, 128), jnp.float32),
    )(pooled.reshape(2, 16, 128), jnp.broadcast_to(b2, (1, 128)))


@jax.jit
def kernel(x, edge_index, batch, W1, b1, W2, b2):
    f32 = jnp.float32
    i32 = jnp.int32

    pad_spread = jnp.arange(E_PAD - E, dtype=i32) % 1024
    src = jnp.concatenate([edge_index[0].astype(i32), N + pad_spread])
    dst = jnp.concatenate([edge_index[1].astype(i32), N + pad_spread])
    dst2d = dst.reshape(E_PAD // 128, 128)

    x_pad = jnp.zeros((N_PAD, D_IN), f32).at[:N].set(x)
    bpad = G + (jnp.arange(N_PAD - N, dtype=i32) % G)
    batch_pad = jnp.concatenate([batch.astype(i32), bpad])
    batch2d = batch_pad.reshape(N_PAD // 128, 128)

    ones_u = jnp.concatenate(
        [jnp.ones((N,), f32), jnp.zeros((N_PAD - N,), f32)])

    accD = _round(ones_u, src, dst2d)

    y, c1s = _matvec(x_pad, W1, W2, b1)
    dinv2, u0_2 = _e1(accD, y.reshape(N_PAD // 128, 128))

    acc1 = _round(u0_2.reshape(N_PAD), src, dst2d)
    u1_2 = _e2(acc1, u0_2, dinv2)

    acc2 = _round(u1_2.reshape(N_PAD), src, dst2d)

    pooled = _pool(acc2, u1_2.reshape(N_PAD), dinv2.reshape(N_PAD),
                   batch2d, c1s.reshape(128))
    out = _final(pooled, b2)
    return out.reshape(G, 1)


# speedup vs baseline: 126.0965x; 1.3708x over previous
import functools

import jax
import jax.numpy as jnp
from jax import lax
from jax.experimental import pallas as pl
from jax.experimental.pallas import tpu as pltpu
from jax.experimental.pallas import tpu_sc as plsc

N = 50000
E = 800000
D_IN = 75
G = 1024

NC = 2
NS = 16
NW = NC * NS

N_PAD = 53248
SLICE_R = N_PAD // NS

E_PAD = 819200
EPW = E_PAD // NW
EROWS = EPW // 128
CHUNK = 3200
NCHUNK = EPW // CHUNK
ROWS = CHUNK // 128

SLICE_P = 2048
NW_P = N_PAD // SLICE_P
PROWS = SLICE_P // 128

G_PAD = 2048
GSLICE = G_PAD // NS

_mesh = plsc.VectorSubcoreMesh(core_axis_name="c", subcore_axis_name="s")
_Z16 = functools.partial(jnp.zeros, (16,), jnp.float32)


def _round_body(u_hbm, src_hbm, dst_hbm, acc_out,
                u_local, src_buf, dst_buf, val_buf, slice_buf, acc_sh,
                sem0, sem1):
    c = lax.axis_index("c")
    s = lax.axis_index("s")
    wid = c * NS + s

    stage = []
    for p in range(8):
        piece = lax.rem(wid + p, 8) * (N_PAD // 8)
        stage.append(pltpu.async_copy(
            u_hbm.at[pl.ds(piece, N_PAD // 8)],
            u_local.at[pl.ds(piece, N_PAD // 8)], sem0))
    stage.append(pltpu.async_copy(
        src_hbm.at[pl.ds(wid * EPW, EPW)], src_buf, sem0))
    stage.append(pltpu.async_copy(
        dst_hbm.at[pl.ds(wid * EROWS, EROWS)], dst_buf, sem0))

    def zbody(i, carry):
        slice_buf[pl.ds(pl.multiple_of(i * 16, 16), 16)] = _Z16()
        return carry

    lax.fori_loop(0, SLICE_R // 16, zbody, 0)
    pltpu.sync_copy(slice_buf, acc_sh.at[pl.ds(s * SLICE_R, SLICE_R)])
    for d in stage:
        d.wait()
    plsc.subcore_barrier()

    descs = [None] * NCHUNK
    for k in range(NCHUNK):
        if k >= 2:
            for d in descs[k - 2]:
                d.wait()
        vb = k % 2

        def gbody(j, carry):
            base = pl.multiple_of(j * 64, 64)
            for t in range(4):
                off = base + t * 16
                idx = src_buf[pl.ds(k * CHUNK + off, 16)]
                val_buf[pl.ds(vb * CHUNK + off, 16)] = plsc.load_gather(
                    u_local, [idx])
            return carry

        lax.fori_loop(0, CHUNK // 64, gbody, 0)

        sem = sem0 if vb == 0 else sem1
        descs[k] = [
            pltpu.async_copy(val_buf.at[pl.ds(vb * CHUNK + j * 128, 128)],
                             acc_sh.at[dst_buf.at[k * ROWS + j]],
                             sem, add=True)
            for j in range(ROWS)
        ]

    for k in (NCHUNK - 2, NCHUNK - 1):
        for d in descs[k]:
            d.wait()

    plsc.subcore_barrier()
    pltpu.sync_copy(acc_sh.at[pl.ds(s * SLICE_R, SLICE_R)],
                    acc_out.at[pl.ds(c * N_PAD + s * SLICE_R, SLICE_R)])


_round = pl.kernel(
    _round_body,
    out_type=jax.ShapeDtypeStruct((NC * N_PAD,), jnp.float32),
    mesh=_mesh,
    compiler_params=pltpu.CompilerParams(needs_layout_passes=False),
    scratch_types=[
        pltpu.VMEM((N_PAD,), jnp.float32),
        pltpu.VMEM((EPW,), jnp.int32),
        pltpu.VMEM((EROWS, 128), jnp.int32),
        pltpu.VMEM((2 * CHUNK,), jnp.float32),
        pltpu.VMEM((SLICE_R,), jnp.float32),
        pltpu.VMEM_SHARED((N_PAD,), jnp.float32),
        pltpu.SemaphoreType.DMA,
        pltpu.SemaphoreType.DMA,
    ],
)


def _degree_body(dst_hbm, acc_out, dst_buf, ones_buf, slice_buf, acc_sh, sem):
    c = lax.axis_index("c")
    s = lax.axis_index("s")
    wid = c * NS + s

    pltpu.sync_copy(dst_hbm.at[pl.ds(wid * EROWS, EROWS)], dst_buf)

    one16 = jnp.ones((16,), jnp.float32)
    for i in range(8):
        ones_buf[pl.ds(i * 16, 16)] = one16

    def zbody(i, carry):
        slice_buf[pl.ds(pl.multiple_of(i * 16, 16), 16)] = _Z16()
        return carry

    lax.fori_loop(0, SLICE_R // 16, zbody, 0)
    pltpu.sync_copy(slice_buf, acc_sh.at[pl.ds(s * SLICE_R, SLICE_R)])
    plsc.subcore_barrier()

    descs = [
        pltpu.async_copy(ones_buf, acc_sh.at[dst_buf.at[j]], sem, add=True)
        for j in range(EROWS)
    ]
    for d in descs:
        d.wait()

    plsc.subcore_barrier()
    pltpu.sync_copy(acc_sh.at[pl.ds(s * SLICE_R, SLICE_R)],
                    acc_out.at[pl.ds(c * N_PAD + s * SLICE_R, SLICE_R)])


_degree = pl.kernel(
    _degree_body,
    out_type=jax.ShapeDtypeStruct((NC * N_PAD,), jnp.float32),
    mesh=_mesh,
    compiler_params=pltpu.CompilerParams(needs_layout_passes=False),
    scratch_types=[
        pltpu.VMEM((EROWS, 128), jnp.int32),
        pltpu.VMEM((128,), jnp.float32),
        pltpu.VMEM((SLICE_R,), jnp.float32),
        pltpu.VMEM_SHARED((N_PAD,), jnp.float32),
        pltpu.SemaphoreType.DMA,
    ],
)


def _round_pool_body(u_hbm, src_hbm, dst_hbm, dinv_hbm, batch_hbm, c1_hbm,
                     pooled_out,
                     u_local, src_buf, dst_buf, val_buf, slice_buf,
                     d_buf, val2_buf, batch_buf, c1_buf, pz_buf,
                     acc_sh, pool_sh, sem0, sem1):
    c = lax.axis_index("c")
    s = lax.axis_index("s")
    wid = c * NS + s

    stage = []
    for p in range(8):
        piece = lax.rem(wid + p, 8) * (N_PAD // 8)
        stage.append(pltpu.async_copy(
            u_hbm.at[pl.ds(piece, N_PAD // 8)],
            u_local.at[pl.ds(piece, N_PAD // 8)], sem0))
    stage.append(pltpu.async_copy(
        src_hbm.at[pl.ds(wid * EPW, EPW)], src_buf, sem0))
    stage.append(pltpu.async_copy(
        dst_hbm.at[pl.ds(wid * EROWS, EROWS)], dst_buf, sem0))

    def zbody(i, carry):
        slice_buf[pl.ds(pl.multiple_of(i * 16, 16), 16)] = _Z16()
        return carry

    lax.fori_loop(0, SLICE_R // 16, zbody, 0)
    pltpu.sync_copy(slice_buf, acc_sh.at[pl.ds(s * SLICE_R, SLICE_R)])

    def pzbody(i, carry):
        pz_buf[pl.ds(pl.multiple_of(i * 16, 16), 16)] = _Z16()
        return carry

    lax.fori_loop(0, GSLICE // 16, pzbody, 0)
    pltpu.sync_copy(pz_buf, pool_sh.at[pl.ds(s * GSLICE, GSLICE)])
    for d in stage:
        d.wait()
    plsc.subcore_barrier()

    descs = [None] * NCHUNK
    for k in range(NCHUNK):
        if k >= 2:
            for d in descs[k - 2]:
                d.wait()
        vb = k % 2

        def gbody(j, carry):
            base = pl.multiple_of(j * 64, 64)
            for t in range(4):
                off = base + t * 16
                idx = src_buf[pl.ds(k * CHUNK + off, 16)]
                val_buf[pl.ds(vb * CHUNK + off, 16)] = plsc.load_gather(
                    u_local, [idx])
            return carry

        lax.fori_loop(0, CHUNK // 64, gbody, 0)

        sem = sem0 if vb == 0 else sem1
        descs[k] = [
            pltpu.async_copy(val_buf.at[pl.ds(vb * CHUNK + j * 128, 128)],
                             acc_sh.at[dst_buf.at[k * ROWS + j]],
                             sem, add=True)
            for j in range(ROWS)
        ]

    for k in (NCHUNK - 2, NCHUNK - 1):
        for d in descs[k]:
            d.wait()

    plsc.subcore_barrier()

    nb = s * SLICE_R
    pltpu.sync_copy(acc_sh.at[pl.ds(nb, SLICE_R)], slice_buf)
    pltpu.sync_copy(dinv_hbm.at[pl.ds(nb, SLICE_R)], d_buf)
    pltpu.sync_copy(batch_hbm.at[pl.ds(s * 32, 32)], batch_buf)
    pltpu.sync_copy(c1_hbm.at[pl.ds(0, 16)], c1_buf)

    c1v = c1_buf[...]
    mf = jnp.where(c == 0, 1.0, 0.0).astype(jnp.float32)

    def ebody(i, carry):
        off = pl.multiple_of(i * 16, 16)
        a = slice_buf[pl.ds(off, 16)]
        d = d_buf[pl.ds(off, 16)]
        u = u_local[pl.ds(nb + off, 16)]
        val2_buf[pl.ds(off, 16)] = d * a + mf * (d * u + c1v)
        return carry

    lax.fori_loop(0, SLICE_R // 16, ebody, 0)

    for j in range(SLICE_R // 128):
        pltpu.sync_copy(val2_buf.at[pl.ds(j * 128, 128)],
                        pool_sh.at[batch_buf.at[j]],
                        add=True)

    plsc.subcore_barrier()
    pltpu.sync_copy(pool_sh.at[pl.ds(s * GSLICE, GSLICE)],
                    pooled_out.at[pl.ds(c * G_PAD + s * GSLICE, GSLICE)])


_round_pool = pl.kernel(
    _round_pool_body,
    out_type=jax.ShapeDtypeStruct((NC * G_PAD,), jnp.float32),
    mesh=_mesh,
    compiler_params=pltpu.CompilerParams(needs_layout_passes=False),
    scratch_types=[
        pltpu.VMEM((N_PAD,), jnp.float32),
        pltpu.VMEM((EPW,), jnp.int32),
        pltpu.VMEM((EROWS, 128), jnp.int32),
        pltpu.VMEM((2 * CHUNK,), jnp.float32),
        pltpu.VMEM((SLICE_R,), jnp.float32),
        pltpu.VMEM((SLICE_R,), jnp.float32),
        pltpu.VMEM((SLICE_R,), jnp.float32),
        pltpu.VMEM((32, 128), jnp.int32),
        pltpu.VMEM((16,), jnp.float32),
        pltpu.VMEM((GSLICE,), jnp.float32),
        pltpu.VMEM_SHARED((N_PAD,), jnp.float32),
        pltpu.VMEM_SHARED((G_PAD,), jnp.float32),
        pltpu.SemaphoreType.DMA,
        pltpu.SemaphoreType.DMA,
    ],
)


def _pool_body(acc2_hbm, u1_hbm, dinv_hbm, batch_hbm, c1_hbm, pooled_out,
               a0_buf, a1_buf, u_buf, d_buf, val_buf, batch_buf, c1_buf,
               pz_buf, pool_sh):
    c = lax.axis_index("c")
    s = lax.axis_index("s")
    wid = c * NS + s

    def zbody(i, carry):
        pz_buf[pl.ds(pl.multiple_of(i * 16, 16), 16)] = _Z16()
        return carry

    lax.fori_loop(0, GSLICE // 16, zbody, 0)
    pltpu.sync_copy(pz_buf, pool_sh.at[pl.ds(s * GSLICE, GSLICE)])

    @pl.when(wid < NW_P)
    def _():
        nb = wid * SLICE_P
        pltpu.sync_copy(acc2_hbm.at[pl.ds(nb, SLICE_P)], a0_buf)
        pltpu.sync_copy(acc2_hbm.at[pl.ds(N_PAD + nb, SLICE_P)], a1_buf)
        pltpu.sync_copy(u1_hbm.at[pl.ds(nb, SLICE_P)], u_buf)
        pltpu.sync_copy(dinv_hbm.at[pl.ds(nb, SLICE_P)], d_buf)
        pltpu.sync_copy(batch_hbm.at[pl.ds(wid * PROWS, PROWS)], batch_buf)
        pltpu.sync_copy(c1_hbm.at[pl.ds(0, 16)], c1_buf)

        c1v = c1_buf[...]

        def ebody(i, carry):
            off = pl.multiple_of(i * 16, 16)
            a0 = a0_buf[pl.ds(off, 16)]
            a1 = a1_buf[pl.ds(off, 16)]
            u = u_buf[pl.ds(off, 16)]
            d = d_buf[pl.ds(off, 16)]
            val_buf[pl.ds(off, 16)] = d * (a0 + a1 + u) + c1v
            return carry

        lax.fori_loop(0, SLICE_P // 16, ebody, 0)

    plsc.subcore_barrier()

    @pl.when(wid < NW_P)
    def _():
        for j in range(PROWS):
            pltpu.sync_copy(val_buf.at[pl.ds(j * 128, 128)],
                            pool_sh.at[batch_buf.at[j]],
                            add=True)

    plsc.subcore_barrier()
    pltpu.sync_copy(pool_sh.at[pl.ds(s * GSLICE, GSLICE)],
                    pooled_out.at[pl.ds(c * G_PAD + s * GSLICE, GSLICE)])


_pool = pl.kernel(
    _pool_body,
    out_type=jax.ShapeDtypeStruct((NC * G_PAD,), jnp.float32),
    mesh=_mesh,
    compiler_params=pltpu.CompilerParams(needs_layout_passes=False),
    scratch_types=[
        pltpu.VMEM((SLICE_P,), jnp.float32),
        pltpu.VMEM((SLICE_P,), jnp.float32),
        pltpu.VMEM((SLICE_P,), jnp.float32),
        pltpu.VMEM((SLICE_P,), jnp.float32),
        pltpu.VMEM((SLICE_P,), jnp.float32),
        pltpu.VMEM((PROWS, 128), jnp.int32),
        pltpu.VMEM((16,), jnp.float32),
        pltpu.VMEM((GSLICE,), jnp.float32),
        pltpu.VMEM_SHARED((G_PAD,), jnp.float32),
    ],
)


def _matvec(x_raw, W1, W2, b1):
    blk = 2000

    def body(x_ref, w1_ref, w2_ref, b1_ref, y_ref, c1_ref):
        hi = lax.Precision.HIGHEST
        w = jnp.dot(w1_ref[...], w2_ref[...], precision=hi,
                    preferred_element_type=jnp.float32)
        y_ref[...] = jnp.dot(x_ref[...], w, precision=hi,
                             preferred_element_type=jnp.float32)
        c1 = jnp.sum(b1_ref[...] * w2_ref[...][:, 0])
        c1_ref[...] = jnp.full((1, 128), c1, jnp.float32)

    return pl.pallas_call(
        body,
        grid=(N // blk,),
        in_specs=[
            pl.BlockSpec((blk, D_IN), lambda i: (i, 0)),
            pl.BlockSpec((D_IN, 128), lambda i: (0, 0)),
            pl.BlockSpec((128, 1), lambda i: (0, 0)),
            pl.BlockSpec((1, 128), lambda i: (0, 0)),
        ],
        out_specs=[
            pl.BlockSpec((blk, 1), lambda i: (i, 0)),
            pl.BlockSpec((1, 128), lambda i: (0, 0)),
        ],
        out_shape=[
            jax.ShapeDtypeStruct((N, 1), jnp.float32),
            jax.ShapeDtypeStruct((1, 128), jnp.float32),
        ],
    )(x_raw, W1, W2, b1.reshape(1, 128))


def _e1(accD, y2):
    def body(a_ref, y_ref, dinv_ref, u0_ref):
        deg = a_ref[...][0] + a_ref[...][1] + 1.0
        dinv = lax.rsqrt(deg)
        dinv_ref[...] = dinv
        u0_ref[...] = dinv * y_ref[...]

    rows = N_PAD // 128
    blk = 32

    return pl.pallas_call(
        body,
        grid=(rows // blk,),
        in_specs=[
            pl.BlockSpec((2, blk, 128), lambda i: (0, i, 0)),
            pl.BlockSpec((blk, 128), lambda i: (i, 0)),
        ],
        out_specs=[
            pl.BlockSpec((blk, 128), lambda i: (i, 0)),
            pl.BlockSpec((blk, 128), lambda i: (i, 0)),
        ],
        out_shape=[
            jax.ShapeDtypeStruct((rows, 128), jnp.float32),
            jax.ShapeDtypeStruct((rows, 128), jnp.float32),
        ],
    )(accD.reshape(2, rows, 128), y2)


def _e2(acc1, u0_2, dinv2):
    def body(a_ref, u_ref, d_ref, u1_ref):
        d = d_ref[...]
        u1_ref[...] = d * d * (a_ref[...][0] + a_ref[...][1] + u_ref[...])

    rows = N_PAD // 128
    blk = 32

    return pl.pallas_call(
        body,
        grid=(rows // blk,),
        in_specs=[
            pl.BlockSpec((2, blk, 128), lambda i: (0, i, 0)),
            pl.BlockSpec((blk, 128), lambda i: (i, 0)),
            pl.BlockSpec((blk, 128), lambda i: (i, 0)),
        ],
        out_specs=pl.BlockSpec((blk, 128), lambda i: (i, 0)),
        out_shape=jax.ShapeDtypeStruct((rows, 128), jnp.float32),
    )(acc1.reshape(2, rows, 128), u0_2, dinv2)


def _final(pooled, b2):
    def body(p_ref, b2_ref, o_ref):
        p = p_ref[...]
        o_ref[...] = p[0, :8] + p[1, :8] + b2_ref[...]

    return pl.pallas_call(
        body,
        in_specs=[
            pl.BlockSpec((2, 16, 128), lambda: (0, 0, 0)),
            pl.BlockSpec((1, 128), lambda: (0, 0)),
        ],
        out_specs=pl.BlockSpec((8, 128), lambda: (0, 0)),
        out_shape=jax.ShapeDtypeStruct((8, 128), jnp.float32),
    )(pooled.reshape(2, 16, 128), jnp.broadcast_to(b2, (1, 128)))


@jax.jit
def kernel(x, edge_index, batch, W1, b1, W2, b2):
    f32 = jnp.float32
    i32 = jnp.int32

    pad_spread = jnp.arange(E_PAD - E, dtype=i32) % 1024
    src = jnp.concatenate([edge_index[0].astype(i32), N + pad_spread])
    dst = jnp.concatenate([edge_index[1].astype(i32), N + pad_spread])
    dst2d = dst.reshape(E_PAD // 128, 128)

    bpad = G + (jnp.arange(N_PAD - N, dtype=i32) % G)
    batch_pad = jnp.concatenate([batch.astype(i32), bpad])
    batch2d = batch_pad.reshape(N_PAD // 128, 128)

    accD = _degree(dst2d)

    y, c1s = _matvec(x, W1, W2, b1)
    y = jnp.zeros((N_PAD,), f32).at[:N].set(y[:, 0])
    dinv2, u0_2 = _e1(accD, y.reshape(N_PAD // 128, 128))

    acc1 = _round(u0_2.reshape(N_PAD), src, dst2d)
    u1_2 = _e2(acc1, u0_2, dinv2)

    batch_al = jnp.pad(batch2d.reshape(16, 26, 128),
                       ((0, 0), (0, 6), (0, 0))).reshape(512, 128)
    pooled = _round_pool(u1_2.reshape(N_PAD), src, dst2d,
                         dinv2.reshape(N_PAD), batch_al, c1s.reshape(128))
    out = _final(pooled, b2)
    return out.reshape(G, 1)
